# baseline (speedup 1.0000x reference)
import functools

import jax
import jax.numpy as jnp
from jax import lax
from jax.experimental import pallas as pl
from jax.experimental.pallas import tpu as pltpu
from jax.experimental.pallas import tpu_sc as plsc

B = 16384
H = 128
NC = 2
NS = 16
NW = NC * NS
CH = 128
NHALF = 2
BH = B // NHALF
ROWS_W = BH // NW
NCH = ROWS_W // CH
BT = 2048
NB = BH // BT


def _gather_body(tt_hbm, tid_hbm, st_hbm, sid_hbm, t_out, s_out,
                 tidx_v, sidx_v, rows_v, dsem):
  wid = lax.axis_index("s") * NC + lax.axis_index("c")
  base = wid * ROWS_W
  pltpu.sync_copy(sid_hbm.at[wid], sidx_v)
  pltpu.sync_copy(tid_hbm.at[wid], tidx_v)
  waits = []
  for c in range(NCH):
    waits.append(pltpu.async_copy(
        st_hbm.at[sidx_v.at[c]], rows_v.at[pl.ds(c * CH, CH)], dsem))
  for w in waits:
    w.wait()
  pltpu.sync_copy(rows_v, s_out.at[pl.ds(base, ROWS_W)])
  waits = []
  for c in range(NCH):
    waits.append(pltpu.async_copy(
        tt_hbm.at[tidx_v.at[c]], rows_v.at[pl.ds(c * CH, CH)], dsem))
  for w in waits:
    w.wait()
  pltpu.sync_copy(rows_v, t_out.at[pl.ds(base, ROWS_W)])


@functools.cache
def _gather_call():
  return pl.kernel(
      _gather_body,
      out_type=[
          jax.ShapeDtypeStruct((BH, H), jnp.float32),
          jax.ShapeDtypeStruct((BH, H), jnp.float32),
      ],
      mesh=plsc.VectorSubcoreMesh(core_axis_name="c", subcore_axis_name="s"),
      scratch_types=[
          pltpu.VMEM((NCH, CH), jnp.int32),
          pltpu.VMEM((NCH, CH), jnp.int32),
          pltpu.VMEM((ROWS_W, H), jnp.float32),
          pltpu.SemaphoreType.DMA,
      ],
  )


def _compute(t_blk, m_blk, xb, w1_ref, b1_ref, w2_ref, b2_ref,
             pcw_ref, pcb_ref, ow_ref, ob_ref):
  f32 = jnp.float32
  wt = ow_ref[0:H, :]
  wm = ow_ref[H:2 * H, :]
  ws = ow_ref[2 * H:3 * H, :]
  wp = ow_ref[3 * H:4 * H, :]
  x = jnp.transpose(xb)
  h = jnp.dot(x, w1_ref[:], preferred_element_type=f32) + b1_ref[:]
  h = h * (1.0 / (1.0 + jnp.exp(-h)))
  s_emb = jnp.dot(h, w2_ref[:], preferred_element_type=f32) + b2_ref[:]
  p_emb = x[:, 4:5] * pcw_ref[:] + pcb_ref[:]
  acc = jnp.dot(t_blk, wt, preferred_element_type=f32)
  acc += jnp.dot(m_blk, wm, preferred_element_type=f32)
  acc += jnp.dot(s_emb, ws, preferred_element_type=f32)
  acc += jnp.dot(p_emb, wp, preferred_element_type=f32)
  return acc + ob_ref[:]


def _tc_body(t_ref, m_ref, xt_ref, w1_ref, b1_ref, w2_ref, b2_ref,
             pcw_ref, pcb_ref, ow_ref, ob_ref, o_ref):
  o_ref[:] = _compute(t_ref[:], m_ref[:], xt_ref[:], w1_ref, b1_ref, w2_ref,
                      b2_ref, pcw_ref, pcb_ref, ow_ref, ob_ref)


def _tc_body2(prev_ref, t_ref, m_ref, xt_ref, w1_ref, b1_ref, w2_ref, b2_ref,
              pcw_ref, pcb_ref, ow_ref, ob_ref, o_ref):
  del prev_ref
  o_ref[:] = _compute(t_ref[:], m_ref[:], xt_ref[:], w1_ref, b1_ref, w2_ref,
                      b2_ref, pcw_ref, pcb_ref, ow_ref, ob_ref)


def _weight_specs():
  rep = lambda i: (0, 0)
  return [
      pl.BlockSpec((8, H), rep),
      pl.BlockSpec((1, H), rep),
      pl.BlockSpec((H, H), rep),
      pl.BlockSpec((1, H), rep),
      pl.BlockSpec((1, H), rep),
      pl.BlockSpec((1, H), rep),
      pl.BlockSpec((4 * H, H), rep),
      pl.BlockSpec((1, H), rep),
  ]


def _tc_half(half_idx, t_h, m_h, xt, weights, out_prev=None):
  off = half_idx * NB
  row = lambda i: (i, 0)
  row_o = lambda i: (i + off, 0)
  col_o = lambda i: (0, i + off)
  base_specs = [
      pl.BlockSpec((BT, H), row),
      pl.BlockSpec((BT, H), row),
      pl.BlockSpec((8, BT), col_o),
  ] + _weight_specs()
  out_shape = jax.ShapeDtypeStruct((B, H), jnp.float32)
  out_spec = pl.BlockSpec((BT, H), row_o)
  if out_prev is None:
    return pl.pallas_call(
        _tc_body, grid=(NB,), in_specs=base_specs, out_specs=out_spec,
        out_shape=out_shape,
    )(t_h, m_h, xt, *weights)
  return pl.pallas_call(
      _tc_body2, grid=(NB,),
      in_specs=[pl.BlockSpec(memory_space=pl.ANY)] + base_specs,
      out_specs=out_spec, out_shape=out_shape,
      input_output_aliases={0: 0},
  )(out_prev, t_h, m_h, xt, *weights)


def kernel(type_ids, semantic_ids, shape_vecs, param_counts, type_table,
           sem_table, shape_W1, shape_b1, shape_W2, shape_b2, pc_W, pc_b,
           out_W, out_b):
  tids = type_ids.astype(jnp.int32).reshape(NHALF, NW, NCH, CH)
  sids = semantic_ids.astype(jnp.int32).reshape(NHALF, NW, NCH, CH)
  xt = (jnp.zeros((8, B), jnp.float32)
        .at[0:4, :].set(shape_vecs.T)
        .at[4, :].set(param_counts))
  w1x = jnp.zeros((8, H), jnp.float32).at[0:4, :H // 2].set(shape_W1)
  b1p = jnp.zeros((1, H), jnp.float32).at[:, :H // 2].set(shape_b1)
  w2p = jnp.zeros((H, H), jnp.float32).at[:H // 2, :].set(shape_W2)
  weights = (w1x, b1p, w2p, shape_b2.reshape(1, H), pc_W,
             pc_b.reshape(1, H), out_W, out_b.reshape(1, H))
  gc = _gather_call()
  t0, m0 = gc(type_table, tids[0], sem_table, sids[0])
  t1, m1 = gc(type_table, tids[1], sem_table, sids[1])
  out = _tc_half(0, t0, m0, xt, weights)
  out = _tc_half(1, t1, m1, xt, weights, out_prev=out)
  return out


# speedup vs baseline: 1.0330x; 1.0330x over previous
import functools

import jax
import jax.numpy as jnp
from jax import lax
from jax.experimental import pallas as pl
from jax.experimental.pallas import tpu as pltpu
from jax.experimental.pallas import tpu_sc as plsc

B = 16384
H = 128
NC = 2
NS = 16
NW = NC * NS
ROWS_W = B // NW
CH = 128
NCH = ROWS_W // CH
PR = 2 * CH

BT = 2048


def _gather_body(tt_hbm, st_hbm, ids_hbm, t_out, s_out,
                 idx_v, b0, b1, b2, s0, s1, s2, ws0, ws1, ws2, ws3):
  wid = lax.axis_index("s") * NC + lax.axis_index("c")
  base = wid * ROWS_W
  pltpu.sync_copy(ids_hbm.at[wid], idx_v)
  ac = pltpu.async_copy

  def gather_pair(tbl, c0, buf, sem):
    return (ac(tbl.at[idx_v.at[c0]], buf.at[pl.ds(0, CH)], sem),
            ac(tbl.at[idx_v.at[c0 + 1]], buf.at[pl.ds(CH, CH)], sem))

  gs0 = gather_pair(st_hbm, 0, b0, s0)
  gs1 = gather_pair(st_hbm, 2, b1, s1)
  for g in gs0:
    g.wait()
  w0 = ac(b0, s_out.at[pl.ds(base, PR)], ws0)
  gt0 = gather_pair(tt_hbm, 4, b2, s2)
  for g in gs1:
    g.wait()
  w1 = ac(b1, s_out.at[pl.ds(base + PR, PR)], ws1)
  for g in gt0:
    g.wait()
  w2 = ac(b2, t_out.at[pl.ds(base, PR)], ws2)
  w0.wait()
  gt1 = gather_pair(tt_hbm, 6, b0, s0)
  for g in gt1:
    g.wait()
  w3 = ac(b0, t_out.at[pl.ds(base + PR, PR)], ws3)
  w1.wait()
  w2.wait()
  w3.wait()


@functools.cache
def _gather_call():
  return pl.kernel(
      _gather_body,
      out_type=[
          jax.ShapeDtypeStruct((B, H), jnp.float32),
          jax.ShapeDtypeStruct((B, H), jnp.float32),
      ],
      mesh=plsc.VectorSubcoreMesh(core_axis_name="c", subcore_axis_name="s"),
      scratch_types=[
          pltpu.VMEM((2 * NCH, CH), jnp.int32),
          pltpu.VMEM((PR, H), jnp.float32),
          pltpu.VMEM((PR, H), jnp.float32),
          pltpu.VMEM((PR, H), jnp.float32),
      ] + [pltpu.SemaphoreType.DMA] * 7,
  )


def _tc_body(t_ref, m_ref, xt_ref, w1_ref, b1_ref, w2_ref, b2_ref,
             pcw_ref, pcb_ref, ow_ref, ob_ref, o_ref):
  f32 = jnp.float32
  wt = ow_ref[0:H, :]
  wm = ow_ref[H:2 * H, :]
  ws = ow_ref[2 * H:3 * H, :]
  wp = ow_ref[3 * H:4 * H, :]
  x = jnp.transpose(xt_ref[:])
  h = jnp.dot(x, w1_ref[:], preferred_element_type=f32) + b1_ref[:]
  h = h * (1.0 / (1.0 + jnp.exp(-h)))
  s_emb = jnp.dot(h, w2_ref[:], preferred_element_type=f32) + b2_ref[:]
  p_emb = x[:, 4:5] * pcw_ref[:] + pcb_ref[:]
  acc = jnp.dot(t_ref[:], wt, preferred_element_type=f32)
  acc += jnp.dot(m_ref[:], wm, preferred_element_type=f32)
  acc += jnp.dot(s_emb, ws, preferred_element_type=f32)
  acc += jnp.dot(p_emb, wp, preferred_element_type=f32)
  o_ref[:] = acc + ob_ref[:]


def _tc_call(t_emb, sem_emb, xt, w1x, b1p, w2p, b2, pc_w, pc_b, out_w,
             out_b, interpret=False):
  nb = B // BT
  row = lambda i: (i, 0)
  col = lambda i: (0, i)
  rep = lambda i: (0, 0)
  return pl.pallas_call(
      _tc_body,
      grid=(nb,),
      in_specs=[
          pl.BlockSpec((BT, H), row),
          pl.BlockSpec((BT, H), row),
          pl.BlockSpec((8, BT), col),
          pl.BlockSpec((8, H), rep),
          pl.BlockSpec((1, H), rep),
          pl.BlockSpec((H, H), rep),
          pl.BlockSpec((1, H), rep),
          pl.BlockSpec((1, H), rep),
          pl.BlockSpec((1, H), rep),
          pl.BlockSpec((4 * H, H), rep),
          pl.BlockSpec((1, H), rep),
      ],
      out_specs=pl.BlockSpec((BT, H), row),
      out_shape=jax.ShapeDtypeStruct((B, H), jnp.float32),
      interpret=interpret,
  )(t_emb, sem_emb, xt, w1x, b1p, w2p, b2, pc_w, pc_b, out_w, out_b)


def kernel(type_ids, semantic_ids, shape_vecs, param_counts, type_table,
           sem_table, shape_W1, shape_b1, shape_W2, shape_b2, pc_W, pc_b,
           out_W, out_b):
  sids = semantic_ids.astype(jnp.int32).reshape(NW, NCH, CH)
  tids = type_ids.astype(jnp.int32).reshape(NW, NCH, CH)
  ids_all = jnp.concatenate([sids, tids], axis=1)
  t_emb, sem_emb = _gather_call()(type_table, sem_table, ids_all)
  xt = (jnp.zeros((8, B), jnp.float32)
        .at[0:4, :].set(shape_vecs.T)
        .at[4, :].set(param_counts))
  w1x = jnp.zeros((8, H), jnp.float32).at[0:4, :H // 2].set(shape_W1)
  b1p = jnp.zeros((1, H), jnp.float32).at[:, :H // 2].set(shape_b1)
  w2p = jnp.zeros((H, H), jnp.float32).at[:H // 2, :].set(shape_W2)
  return _tc_call(t_emb, sem_emb, xt, w1x, b1p,
                  w2p, shape_b2.reshape(1, H), pc_W, pc_b.reshape(1, H),
                  out_W, out_b.reshape(1, H))


# speedup vs baseline: 1.0512x; 1.0176x over previous
import functools

import jax
import jax.numpy as jnp
from jax import lax
from jax.experimental import pallas as pl
from jax.experimental.pallas import tpu as pltpu
from jax.experimental.pallas import tpu_sc as plsc

B = 16384
H = 128
NC = 2
NS = 16
NW = NC * NS
ROWS_W = B // NW
CH = 128
NCH = ROWS_W // CH

BT = 2048


def _gather_body(tt_hbm, tid_hbm, st_hbm, sid_hbm, t_out, s_out,
                 tidx_v, sidx_v, rows_v, dsem):
  wid = lax.axis_index("s") * NC + lax.axis_index("c")
  base = wid * ROWS_W
  pltpu.sync_copy(sid_hbm.at[wid], sidx_v)
  pltpu.sync_copy(tid_hbm.at[wid], tidx_v)
  waits = []
  for c in range(NCH):
    waits.append(pltpu.async_copy(
        st_hbm.at[sidx_v.at[c]], rows_v.at[pl.ds(c * CH, CH)], dsem))
  for w in waits:
    w.wait()
  pltpu.sync_copy(rows_v, s_out.at[pl.ds(base, ROWS_W)])
  waits = []
  for c in range(NCH):
    waits.append(pltpu.async_copy(
        tt_hbm.at[tidx_v.at[c]], rows_v.at[pl.ds(c * CH, CH)], dsem))
  for w in waits:
    w.wait()
  pltpu.sync_copy(rows_v, t_out.at[pl.ds(base, ROWS_W)])


@functools.cache
def _gather_call():
  return pl.kernel(
      _gather_body,
      out_type=[
          jax.ShapeDtypeStruct((B, H), jnp.float32),
          jax.ShapeDtypeStruct((B, H), jnp.float32),
      ],
      mesh=plsc.VectorSubcoreMesh(core_axis_name="c", subcore_axis_name="s"),
      scratch_types=[
          pltpu.VMEM((NCH, CH), jnp.int32),
          pltpu.VMEM((NCH, CH), jnp.int32),
          pltpu.VMEM((ROWS_W, H), jnp.float32),
          pltpu.SemaphoreType.DMA,
      ],
  )


def _tc_body(t_ref, m_ref, xt_ref, w1_ref, b1_ref, w2_ref, b2_ref,
             pcw_ref, pcb_ref, ow_ref, ob_ref, o_ref):
  f32 = jnp.float32
  wt = ow_ref[0:H, :]
  wm = ow_ref[H:2 * H, :]
  ws = ow_ref[2 * H:3 * H, :]
  wp = ow_ref[3 * H:4 * H, :]
  x = jnp.transpose(xt_ref[:])
  h = jnp.dot(x, w1_ref[:], preferred_element_type=f32) + b1_ref[:]
  h = h * (1.0 / (1.0 + jnp.exp(-h)))
  s_emb = jnp.dot(h, w2_ref[:], preferred_element_type=f32) + b2_ref[:]
  p_emb = x[:, 4:5] * pcw_ref[:] + pcb_ref[:]
  acc = jnp.dot(t_ref[:], wt, preferred_element_type=f32)
  acc += jnp.dot(m_ref[:], wm, preferred_element_type=f32)
  acc += jnp.dot(s_emb, ws, preferred_element_type=f32)
  acc += jnp.dot(p_emb, wp, preferred_element_type=f32)
  o_ref[:] = acc + ob_ref[:]


def _tc_call(t_emb, sem_emb, xt, w1x, b1p, w2p, b2, pc_w, pc_b, out_w,
             out_b, interpret=False):
  nb = B // BT
  row = lambda i: (i, 0)
  col = lambda i: (0, i)
  rep = lambda i: (0, 0)
  return pl.pallas_call(
      _tc_body,
      grid=(nb,),
      in_specs=[
          pl.BlockSpec((BT, H), row),
          pl.BlockSpec((BT, H), row),
          pl.BlockSpec((8, BT), col),
          pl.BlockSpec((8, H), rep),
          pl.BlockSpec((1, H), rep),
          pl.BlockSpec((H, H), rep),
          pl.BlockSpec((1, H), rep),
          pl.BlockSpec((1, H), rep),
          pl.BlockSpec((1, H), rep),
          pl.BlockSpec((4 * H, H), rep),
          pl.BlockSpec((1, H), rep),
      ],
      out_specs=pl.BlockSpec((BT, H), row),
      out_shape=jax.ShapeDtypeStruct((B, H), jnp.float32),
      interpret=interpret,
  )(t_emb, sem_emb, xt, w1x, b1p, w2p, b2, pc_w, pc_b, out_w, out_b)


def kernel(type_ids, semantic_ids, shape_vecs, param_counts, type_table,
           sem_table, shape_W1, shape_b1, shape_W2, shape_b2, pc_W, pc_b,
           out_W, out_b):
  tids = type_ids.astype(jnp.int32).reshape(NW, NCH, CH)
  sids = semantic_ids.astype(jnp.int32).reshape(NW, NCH, CH)
  t_emb, sem_emb = _gather_call()(type_table, tids, sem_table, sids)
  xt = (jnp.zeros((8, B), jnp.float32)
        .at[0:4, :].set(shape_vecs.T)
        .at[4, :].set(param_counts))
  w1x = jnp.zeros((8, H), jnp.float32).at[0:4, :H // 2].set(shape_W1)
  b1p = jnp.zeros((1, H), jnp.float32).at[:, :H // 2].set(shape_b1)
  w2p = jnp.zeros((H, H), jnp.float32).at[:H // 2, :].set(shape_W2)
  return _tc_call(t_emb, sem_emb, xt, w1x, b1p,
                  w2p, shape_b2.reshape(1, H), pc_W, pc_b.reshape(1, H),
                  out_W, out_b.reshape(1, H))
